# baseline (speedup 1.0000x reference)
import jax
import jax.numpy as jnp
from jax.experimental import pallas as pl
from jax.experimental.pallas import tpu as pltpu


def _detect_body(x_ref, w_ref, b_ref, o_ref):
    acc = jax.lax.dot_general(
        x_ref[0], w_ref[...],
        dimension_numbers=(((0,), (0,)), ((), ())),
        preferred_element_type=jnp.float32,
    )
    o_ref[0] = acc + b_ref[...]


def _detect_level(x, w, b, tile_hw):
    bsz, c, h, wdim = x.shape
    o = w.shape[0]
    hw = h * wdim
    xr = x.reshape(bsz, c, hw)
    wt = w.reshape(o, c).T
    br = b.reshape(1, o)
    grid = (bsz, hw // tile_hw)
    out = pl.pallas_call(
        _detect_body,
        grid=grid,
        in_specs=[
            pl.BlockSpec((1, c, tile_hw), lambda bi, ti: (bi, 0, ti)),
            pl.BlockSpec((c, o), lambda bi, ti: (0, 0)),
            pl.BlockSpec((1, o), lambda bi, ti: (0, 0)),
        ],
        out_specs=pl.BlockSpec((1, tile_hw, o), lambda bi, ti: (bi, ti, 0)),
        out_shape=jax.ShapeDtypeStruct((bsz, hw, o), jnp.float32),
        compiler_params=pltpu.CompilerParams(
            dimension_semantics=("parallel", "parallel"),
            allow_input_fusion=[False, True, False],
        ),
    )(xr, wt, br)
    return out.reshape(bsz, h, wdim, o)


def kernel(x0, x1, w0, b0, w1, b1, export):
    y0 = _detect_level(x0, w0, b0, tile_hw=1024)
    y1 = _detect_level(x1, w1, b1, tile_hw=1024)
    return (y0, y1)


# speedup vs baseline: 1.1052x; 1.1052x over previous
import jax
import jax.numpy as jnp
from jax.experimental import pallas as pl
from jax.experimental.pallas import tpu as pltpu


def _detect_body(x_ref, w_ref, b_ref, o_ref):
    acc = jax.lax.dot_general(
        x_ref[0], w_ref[...],
        dimension_numbers=(((0,), (0,)), ((), ())),
        preferred_element_type=jnp.float32,
    )
    o_ref[0] = acc + b_ref[...]


def _detect_level(x, w, b, tile_hw):
    bsz, c, h, wdim = x.shape
    o = w.shape[0]
    hw = h * wdim
    xr = x.reshape(bsz, c, hw)
    wt = w.reshape(o, c).T
    br = b.reshape(1, o)
    grid = (bsz, hw // tile_hw)
    out = pl.pallas_call(
        _detect_body,
        grid=grid,
        in_specs=[
            pl.BlockSpec((1, c, tile_hw), lambda bi, ti: (bi, 0, ti)),
            pl.BlockSpec((c, o), lambda bi, ti: (0, 0)),
            pl.BlockSpec((1, o), lambda bi, ti: (0, 0)),
        ],
        out_specs=pl.BlockSpec((1, tile_hw, o), lambda bi, ti: (bi, ti, 0)),
        out_shape=jax.ShapeDtypeStruct((bsz, hw, o), jnp.float32),
        compiler_params=pltpu.CompilerParams(
            dimension_semantics=("parallel", "parallel"),
        ),
    )(xr, wt, br)
    return out.reshape(bsz, h, wdim, o)


def kernel(x0, x1, w0, b0, w1, b1, export):
    y0 = _detect_level(x0, w0, b0, tile_hw=1024)
    y1 = _detect_level(x1, w1, b1, tile_hw=1024)
    return (y0, y1)
